# baseline (speedup 1.0000x reference)
import jax
import jax.numpy as jnp
from jax import lax
from jax.experimental import pallas as pl
from jax.experimental.pallas import tpu as pltpu
from jax.experimental.pallas import tpu_sc as plsc

NC = 2
NS = 16
NW = NC * NS
L = 16

B, T, D = 16, 2048, 512
ROWS_W = T // NW


def _sc_trim(seq_hbm, pe_hbm, out_hbm, pe_v, seq_v):
    wid = lax.axis_index("s") * NC + lax.axis_index("c")
    row0 = wid * ROWS_W

    pltpu.sync_copy(pe_hbm.at[0, pl.ds(row0, ROWS_W)], pe_v)

    @pl.loop(0, B)
    def _batch(b):
        pltpu.sync_copy(seq_hbm.at[b, pl.ds(row0, ROWS_W)], seq_v)

        @pl.loop(0, ROWS_W)
        def _row(r):
            @plsc.parallel_loop(0, D, step=L, unroll=8)
            def _col(c):
                seq_v[r, pl.ds(c, L)] = seq_v[r, pl.ds(c, L)] * 2.0 + pe_v[r, pl.ds(c, L)]

        pltpu.sync_copy(seq_v, out_hbm.at[b, 0, pl.ds(row0, ROWS_W)])


def kernel(seq, times, pe):
    del times
    mesh = plsc.VectorSubcoreMesh(core_axis_name="c", subcore_axis_name="s")
    out = pl.kernel(
        _sc_trim,
        out_type=jax.ShapeDtypeStruct((B, 1, T, D), jnp.float32),
        mesh=mesh,
        scratch_types=[
            pltpu.VMEM((ROWS_W, D), jnp.float32),
            pltpu.VMEM((ROWS_W, D), jnp.float32),
        ],
    )(seq, pe)
    mask = jnp.ones((B, 1), dtype=bool)
    return (out, mask)


# speedup vs baseline: 2.8001x; 2.8001x over previous
import jax
import jax.numpy as jnp
from jax import lax
from jax.experimental import pallas as pl
from jax.experimental.pallas import tpu as pltpu
from jax.experimental.pallas import tpu_sc as plsc

NC = 2
NS = 16
NW = NC * NS
L = 16

B, T, D = 16, 2048, 512
ROWS_W = T // NW


def _sc_trim(seq_hbm, pe_hbm, out_hbm, pe_v, seq_v):
    wid = lax.axis_index("s") * NC + lax.axis_index("c")
    row0 = wid * ROWS_W

    pltpu.sync_copy(pe_hbm.at[0, pl.ds(row0, ROWS_W)], pe_v)

    @pl.loop(0, B)
    def _batch(b):
        pltpu.sync_copy(seq_hbm.at[b, pl.ds(row0, ROWS_W)], seq_v)

        @pl.loop(0, ROWS_W)
        def _row(r):
            @plsc.parallel_loop(0, D, step=L, unroll=8)
            def _col(c):
                seq_v[r, pl.ds(c, L)] = seq_v[r, pl.ds(c, L)] * 2.0 + pe_v[r, pl.ds(c, L)]

        pltpu.sync_copy(seq_v, out_hbm.at[b, 0, pl.ds(row0, ROWS_W)])


def _tc_trim(seq_ref, pe_ref, out_ref):
    out_ref[:, 0] = seq_ref[...] * 2.0 + pe_ref[...]


BATCHES_PER_BLOCK = 2


def kernel(seq, times, pe):
    del times
    b, t, d = seq.shape
    pe2 = pe[0]
    bb = BATCHES_PER_BLOCK

    out = pl.pallas_call(
        _tc_trim,
        grid=(b // bb,),
        in_specs=[
            pl.BlockSpec((bb, t, d), lambda bi: (bi, 0, 0)),
            pl.BlockSpec((t, d), lambda bi: (0, 0)),
        ],
        out_specs=pl.BlockSpec((bb, 1, t, d), lambda bi: (bi, 0, 0, 0)),
        out_shape=jax.ShapeDtypeStruct((b, 1, t, d), seq.dtype),
    )(seq, pe2)

    mask = jnp.ones((b, 1), dtype=bool)
    return (out, mask)
